# baseline (speedup 1.0000x reference)
import jax
import jax.numpy as jnp
from jax.experimental import pallas as pl
from jax.experimental.pallas import tpu as pltpu
from jax.experimental.pallas import tpu_sc as plsc

_GATHER_WINDOW = 128
_VOCAB_TILE = 1024


def _sc_gather(table, indices):
    b = indices.shape[0]
    d = table.shape[1]
    idx2d = indices.reshape(1, b)
    mesh = plsc.VectorSubcoreMesh(core_axis_name="core", subcore_axis_name="subcore")

    @pl.kernel(out_type=jax.ShapeDtypeStruct((b, d), table.dtype), mesh=mesh)
    def gather_kernel(x_hbm, i_hbm, o_hbm):
        def body(i_vmem, o_vmem):
            pltpu.sync_copy(x_hbm.at[i_vmem.at[0]], o_vmem)

        pltpu.emit_pipeline(
            body,
            grid=(b // _GATHER_WINDOW,),
            in_specs=[pl.BlockSpec((1, _GATHER_WINDOW), index_map=lambda i: (0, i))],
            out_specs=[pl.BlockSpec((_GATHER_WINDOW, d), index_map=lambda i: (i, 0))],
            core_axis_name=("core", "subcore"),
            dimension_semantics=(pltpu.PARALLEL,),
        )(i_hbm, o_hbm)

    return gather_kernel(table, idx2d)


def _select_body(d, packed_ref, parity_ref, o_ref):
    lo = packed_ref[:, :d]
    hi = packed_ref[:, d:]
    o_ref[...] = jnp.where(parity_ref[...] != 0, hi, lo).astype(jnp.bfloat16)


def _matmul_body(e_ref, c_ref, o_ref):
    o_ref[...] = jax.lax.dot_general(
        e_ref[...].astype(jnp.bfloat16),
        c_ref[...],
        dimension_numbers=(((1,), (1,)), ((), ())),
        preferred_element_type=jnp.float32,
        precision=jax.lax.Precision.DEFAULT,
    )


def kernel(center_words, in_emb, out_emb):
    b = center_words.shape[0]
    v, d = out_emb.shape

    packed = _sc_gather(in_emb.reshape(v // 2, 2 * d), center_words >> 1)
    parity = (center_words & 1).reshape(b, 1)

    center_bf = pl.pallas_call(
        lambda p_ref, q_ref, o_ref: _select_body(d, p_ref, q_ref, o_ref),
        in_specs=[
            pl.BlockSpec((b, 2 * d), lambda: (0, 0)),
            pl.BlockSpec((b, 1), lambda: (0, 0)),
        ],
        out_specs=pl.BlockSpec((b, d), lambda: (0, 0)),
        out_shape=jax.ShapeDtypeStruct((b, d), jnp.bfloat16),
    )(packed, parity)

    num_tiles = pl.cdiv(v, _VOCAB_TILE)
    scores_t = pl.pallas_call(
        _matmul_body,
        grid=(num_tiles,),
        in_specs=[
            pl.BlockSpec((_VOCAB_TILE, d), lambda j: (j, 0)),
            pl.BlockSpec((b, d), lambda j: (0, 0)),
        ],
        out_specs=pl.BlockSpec((_VOCAB_TILE, b), lambda j: (j, 0)),
        out_shape=jax.ShapeDtypeStruct((v, b), jnp.float32),
    )(out_emb, center_bf)
    return scores_t.T


# speedup vs baseline: 1.0219x; 1.0219x over previous
import jax
import jax.numpy as jnp
from jax.experimental import pallas as pl
from jax.experimental.pallas import tpu as pltpu
from jax.experimental.pallas import tpu_sc as plsc

_GATHER_WINDOW = 128
_VOCAB_TILE = 1024


def _sc_gather(table, indices):
    nw, w = indices.shape
    d = table.shape[1]
    mesh = plsc.VectorSubcoreMesh(core_axis_name="core", subcore_axis_name="subcore")

    @pl.kernel(out_type=jax.ShapeDtypeStruct((nw * w, d), table.dtype), mesh=mesh)
    def gather_kernel(x_hbm, i_hbm, o_hbm):
        def body(i_vmem, o_vmem):
            pltpu.sync_copy(x_hbm.at[i_vmem.at[0]], o_vmem)

        pltpu.emit_pipeline(
            body,
            grid=(nw,),
            in_specs=[pl.BlockSpec((1, w), index_map=lambda i: (i, 0))],
            out_specs=[pl.BlockSpec((w, d), index_map=lambda i: (i, 0))],
            core_axis_name=("core", "subcore"),
            dimension_semantics=(pltpu.PARALLEL,),
        )(i_hbm, o_hbm)

    return gather_kernel(table, indices)


def _matmul_body(d, e_ref, c_ref, o_ref, cbf_ref):
    @pl.when(pl.program_id(0) == 0)
    def _():
        cbf_ref[...] = c_ref[:, :d].astype(jnp.bfloat16)

    o_ref[...] = jax.lax.dot_general(
        e_ref[...].astype(jnp.bfloat16),
        cbf_ref[...],
        dimension_numbers=(((1,), (1,)), ((), ())),
        preferred_element_type=jnp.float32,
        precision=jax.lax.Precision.DEFAULT,
    )


def kernel(center_words, in_emb, out_emb):
    b = center_words.shape[0]
    v, d = out_emb.shape

    in_pad = jnp.pad(in_emb, ((0, 0), (0, 128 - d)))
    center_pack = _sc_gather(in_pad, center_words.reshape(-1, _GATHER_WINDOW))

    num_tiles = pl.cdiv(v, _VOCAB_TILE)
    scores_t = pl.pallas_call(
        lambda e_ref, c_ref, o_ref, cbf_ref: _matmul_body(d, e_ref, c_ref, o_ref, cbf_ref),
        grid=(num_tiles,),
        in_specs=[
            pl.BlockSpec((_VOCAB_TILE, d), lambda j: (j, 0)),
            pl.BlockSpec((b, 128), lambda j: (0, 0)),
        ],
        out_specs=pl.BlockSpec((_VOCAB_TILE, b), lambda j: (j, 0)),
        out_shape=jax.ShapeDtypeStruct((v, b), jnp.float32),
        scratch_shapes=[pltpu.VMEM((b, d), jnp.bfloat16)],
    )(out_emb, center_pack)
    return scores_t.T
